# baseline (speedup 1.0000x reference)
import jax
import jax.numpy as jnp
from jax import lax
from jax.experimental import pallas as pl
from jax.experimental.pallas import tpu as pltpu
from jax.experimental.pallas import tpu_sc as plsc

N_NODES = 10000
DIM = 128
HDIM = DIM // 2
N_EDGES = 320000

NC, NS = 2, 16
CHUNK = 64
CPT = 320
PAD_EDGES = NS * CPT * CHUNK
PHASE = 80
ROWS_PER_SUB = 624
HVREGS = HDIM // 16
NBUF = 4


def _spmm_body(h_hbm, row_hbm, col_hbm, w_hbm, out_hbm,
               acc, hspm, colv, ridxv, wv, rows, scaled, gsems, ssems):
    c = lax.axis_index("c")
    s = lax.axis_index("s")
    base_row = pl.multiple_of(s * ROWS_PER_SUB, 8)
    hsrc = h_hbm.at[c]

    for phase in range(CPT // PHASE):
        pbase = s * CPT + phase * PHASE
        pltpu.sync_copy(col_hbm.at[pl.ds(pbase, PHASE)], colv)
        pltpu.sync_copy(row_hbm.at[pl.ds(pbase, PHASE)], ridxv)
        pltpu.sync_copy(w_hbm.at[pl.ds(pbase, PHASE)], wv)

        if phase == 0:
            pltpu.sync_copy(hsrc.at[pl.ds(base_row, 640), :],
                            hspm.at[pl.ds(base_row, 640), :])
            def _zero_row(e, _):
                for j in range(HVREGS):
                    scaled[0, e, 16 * j:16 * (j + 1)] = jnp.zeros(
                        (16,), jnp.float32)
                return 0
            lax.fori_loop(0, CHUNK, _zero_row, 0)
            for k in range(10):
                pltpu.sync_copy(scaled.at[0, pl.ds(0, CHUNK), :],
                                acc.at[pl.ds(base_row + CHUNK * k, CHUNK), :])
            plsc.subcore_barrier()

        for b in range(NBUF):
            src_tbl = hsrc if b % 2 == 0 else hspm
            pltpu.async_copy(src_tbl.at[colv.at[b]], rows.at[b], gsems.at[b])

        def _step(t, b):
            i = NBUF * t + b
            sb = b % 2
            src_tbl = hsrc if b % 2 == 0 else hspm
            pltpu.make_async_copy(src_tbl.at[colv.at[i]], rows.at[b],
                                  gsems.at[b]).wait()

            @pl.when(i >= 2)
            def _():
                pltpu.make_async_copy(scaled.at[sb], acc.at[ridxv.at[i - 2]],
                                      ssems.at[sb]).wait()

            def _scale_group(g, _):
                wg = wv[i, pl.ds(g * 16, 16)]
                for e in range(16):
                    wspl = jnp.broadcast_to(wg[e], (16,))
                    idx = g * 16 + e
                    for j in range(HVREGS):
                        sl = slice(16 * j, 16 * (j + 1))
                        scaled[sb, idx, sl] = rows[b, idx, sl] * wspl
                return 0
            lax.fori_loop(0, CHUNK // 16, _scale_group, 0)
            pltpu.async_copy(scaled.at[sb], acc.at[ridxv.at[i]],
                             ssems.at[sb], add=True)

            @pl.when(i + NBUF < PHASE)
            def _():
                pltpu.async_copy(src_tbl.at[colv.at[i + NBUF]], rows.at[b],
                                 gsems.at[b])

        def _loop_body(t, _):
            for b in range(NBUF):
                _step(t, b)
            return 0
        lax.fori_loop(0, PHASE // NBUF, _loop_body, 0)

        for i in (PHASE - 2, PHASE - 1):
            pltpu.make_async_copy(scaled.at[i % 2], acc.at[ridxv.at[i]],
                                  ssems.at[i % 2]).wait()

    plsc.subcore_barrier()

    col_base = pl.multiple_of(c * HDIM, 8)

    @pl.when(s < NS - 1)
    def _():
        pltpu.sync_copy(acc.at[pl.ds(base_row, ROWS_PER_SUB), :],
                        out_hbm.at[pl.ds(base_row, ROWS_PER_SUB),
                                   pl.ds(col_base, HDIM)])

    @pl.when(s == NS - 1)
    def _():
        last = (NS - 1) * ROWS_PER_SUB
        pltpu.sync_copy(acc.at[pl.ds(last, N_NODES - last), :],
                        out_hbm.at[pl.ds(last, N_NODES - last),
                                   pl.ds(col_base, HDIM)])


@jax.jit
def _spmm(h2, row, col, w):
    mesh = plsc.VectorSubcoreMesh(core_axis_name="c", subcore_axis_name="s")
    run = pl.kernel(
        _spmm_body,
        out_type=jax.ShapeDtypeStruct((N_NODES, DIM), jnp.float32),
        mesh=mesh,
        compiler_params=pltpu.CompilerParams(use_tc_tiling_on_sc=False),
        scratch_types=[
            pltpu.VMEM_SHARED((N_NODES, HDIM), jnp.float32),
            pltpu.VMEM_SHARED((N_NODES, HDIM), jnp.float32),
            pltpu.VMEM((PHASE, CHUNK), jnp.int32),
            pltpu.VMEM((PHASE, CHUNK), jnp.int32),
            pltpu.VMEM((PHASE, CHUNK), jnp.float32),
            pltpu.VMEM((NBUF, CHUNK, HDIM), jnp.float32),
            pltpu.VMEM((2, CHUNK, HDIM), jnp.float32),
            pltpu.SemaphoreType.DMA((NBUF,)),
            pltpu.SemaphoreType.DMA((2,)),
        ],
    )
    return run(h2, row, col, w)


ROW_BLK = 5000


def _mm_body(x_ref, w_ref, b_ref, o_ref, *, relu_in):
    xin = x_ref[...]
    if relu_in:
        xin = jnp.maximum(xin, 0.0)
    h = jnp.dot(xin, w_ref[...], preferred_element_type=jnp.float32) \
        + b_ref[...]
    o_ref[0] = h[:, :HDIM]
    o_ref[1] = h[:, HDIM:]


def _mm(x, W, b, relu_in):
    import functools
    return pl.pallas_call(
        functools.partial(_mm_body, relu_in=relu_in),
        grid=(N_NODES // ROW_BLK,),
        in_specs=[
            pl.BlockSpec((ROW_BLK, DIM), lambda i: (i, 0)),
            pl.BlockSpec((DIM, DIM), lambda i: (0, 0)),
            pl.BlockSpec((1, DIM), lambda i: (0, 0)),
        ],
        out_specs=pl.BlockSpec((NC, ROW_BLK, HDIM), lambda i: (0, i, 0)),
        out_shape=jax.ShapeDtypeStruct((NC, N_NODES, HDIM), jnp.float32),
    )(x, W, b.reshape(1, DIM))


def kernel(x, edge_index, edge_weight, W1, b1, W2, b2):
    pad = PAD_EDGES - N_EDGES
    spread = (jnp.arange(pad, dtype=jnp.int32) * 37) % N_NODES
    row = jnp.concatenate(
        [edge_index[0].astype(jnp.int32), spread]).reshape(-1, CHUNK)
    col = jnp.concatenate(
        [edge_index[1].astype(jnp.int32), spread]).reshape(-1, CHUNK)
    w = jnp.concatenate(
        [edge_weight.astype(jnp.float32),
         jnp.zeros((pad,), jnp.float32)]).reshape(-1, CHUNK)

    h1 = _mm(x, W1, b1, relu_in=False)
    p1 = _spmm(h1, row, col, w)
    h2 = _mm(p1, W2, b2, relu_in=True)
    return _spmm(h2, row, col, w)


# speedup vs baseline: 1.3895x; 1.3895x over previous
import jax
import jax.numpy as jnp
from jax import lax
from jax.experimental import pallas as pl
from jax.experimental.pallas import tpu as pltpu
from jax.experimental.pallas import tpu_sc as plsc

N_NODES = 10000
DIM = 128
HDIM = DIM // 2
N_EDGES = 320000

NC, NS = 2, 16
CHUNK = 64
CPT = 320
PAD_EDGES = NS * CPT * CHUNK
PHASE = 160
ROWS_PER_SUB = 624
HVREGS = HDIM // 16
NBUF = 8


def _spmm_body(h_hbm, row_hbm, col_hbm, w_hbm, out_hbm,
               acc, colv, ridxv, wv, rows, scaled, gsems, ssems):
    c = lax.axis_index("c")
    s = lax.axis_index("s")
    base_row = pl.multiple_of(s * ROWS_PER_SUB, 8)
    hsrc = h_hbm.at[c]

    for phase in range(CPT // PHASE):
        pbase = s * CPT + phase * PHASE
        pltpu.sync_copy(col_hbm.at[pl.ds(pbase, PHASE)], colv)
        pltpu.sync_copy(row_hbm.at[pl.ds(pbase, PHASE)], ridxv)
        pltpu.sync_copy(w_hbm.at[pl.ds(pbase, PHASE)], wv)

        for b in range(NBUF):
            pltpu.async_copy(hsrc.at[colv.at[b]], rows.at[b], gsems.at[b])

        if phase == 0:
            def _zero_row(e, _):
                for j in range(HVREGS):
                    scaled[0, e, 16 * j:16 * (j + 1)] = jnp.zeros(
                        (16,), jnp.float32)
                return 0
            lax.fori_loop(0, CHUNK, _zero_row, 0)
            for k in range(10):
                pltpu.sync_copy(scaled.at[0, pl.ds(0, CHUNK), :],
                                acc.at[pl.ds(base_row + CHUNK * k, CHUNK), :])
            plsc.subcore_barrier()

        def _step(t, b):
            i = NBUF * t + b
            sb = b % 2
            pltpu.make_async_copy(hsrc.at[colv.at[i]], rows.at[b],
                                  gsems.at[b]).wait()

            @pl.when(i >= 2)
            def _():
                pltpu.make_async_copy(scaled.at[sb], acc.at[ridxv.at[i - 2]],
                                      ssems.at[sb]).wait()

            def _scale_group(g, _):
                wg = wv[i, pl.ds(g * 16, 16)]
                for e in range(16):
                    wspl = jnp.broadcast_to(wg[e], (16,))
                    idx = g * 16 + e
                    for j in range(HVREGS):
                        sl = slice(16 * j, 16 * (j + 1))
                        scaled[sb, idx, sl] = rows[b, idx, sl] * wspl
                return 0
            lax.fori_loop(0, CHUNK // 16, _scale_group, 0)
            pltpu.async_copy(scaled.at[sb], acc.at[ridxv.at[i]],
                             ssems.at[sb], add=True)

            @pl.when(i + NBUF < PHASE)
            def _():
                pltpu.async_copy(hsrc.at[colv.at[i + NBUF]], rows.at[b],
                                 gsems.at[b])

        def _loop_body(t, _):
            for b in range(NBUF):
                _step(t, b)
            return 0
        lax.fori_loop(0, PHASE // NBUF, _loop_body, 0)

        for i in (PHASE - 2, PHASE - 1):
            pltpu.make_async_copy(scaled.at[i % 2], acc.at[ridxv.at[i]],
                                  ssems.at[i % 2]).wait()

    plsc.subcore_barrier()

    col_base = pl.multiple_of(c * HDIM, 8)

    @pl.when(s < NS - 1)
    def _():
        pltpu.sync_copy(acc.at[pl.ds(base_row, ROWS_PER_SUB), :],
                        out_hbm.at[pl.ds(base_row, ROWS_PER_SUB),
                                   pl.ds(col_base, HDIM)])

    @pl.when(s == NS - 1)
    def _():
        last = (NS - 1) * ROWS_PER_SUB
        pltpu.sync_copy(acc.at[pl.ds(last, N_NODES - last), :],
                        out_hbm.at[pl.ds(last, N_NODES - last),
                                   pl.ds(col_base, HDIM)])


@jax.jit
def _spmm(h2, row, col, w):
    mesh = plsc.VectorSubcoreMesh(core_axis_name="c", subcore_axis_name="s")
    run = pl.kernel(
        _spmm_body,
        out_type=jax.ShapeDtypeStruct((N_NODES, DIM), jnp.float32),
        mesh=mesh,
        compiler_params=pltpu.CompilerParams(use_tc_tiling_on_sc=False),
        scratch_types=[
            pltpu.VMEM_SHARED((N_NODES, HDIM), jnp.float32),
            pltpu.VMEM((PHASE, CHUNK), jnp.int32),
            pltpu.VMEM((PHASE, CHUNK), jnp.int32),
            pltpu.VMEM((PHASE, CHUNK), jnp.float32),
            pltpu.VMEM((NBUF, CHUNK, HDIM), jnp.float32),
            pltpu.VMEM((2, CHUNK, HDIM), jnp.float32),
            pltpu.SemaphoreType.DMA((NBUF,)),
            pltpu.SemaphoreType.DMA((2,)),
        ],
    )
    return run(h2, row, col, w)


ROW_BLK = 10000


def _mm_body(x_ref, w_ref, b_ref, o_ref, *, relu_in):
    xin = x_ref[...]
    if relu_in:
        xin = jnp.maximum(xin, 0.0)
    h = jnp.dot(xin, w_ref[...], preferred_element_type=jnp.float32) \
        + b_ref[...]
    o_ref[0] = h[:, :HDIM]
    o_ref[1] = h[:, HDIM:]


def _mm(x, W, b, relu_in):
    import functools
    return pl.pallas_call(
        functools.partial(_mm_body, relu_in=relu_in),
        grid=(N_NODES // ROW_BLK,),
        in_specs=[
            pl.BlockSpec((ROW_BLK, DIM), lambda i: (i, 0)),
            pl.BlockSpec((DIM, DIM), lambda i: (0, 0)),
            pl.BlockSpec((1, DIM), lambda i: (0, 0)),
        ],
        out_specs=pl.BlockSpec((NC, ROW_BLK, HDIM), lambda i: (0, i, 0)),
        out_shape=jax.ShapeDtypeStruct((NC, N_NODES, HDIM), jnp.float32),
    )(x, W, b.reshape(1, DIM))


def kernel(x, edge_index, edge_weight, W1, b1, W2, b2):
    pad = PAD_EDGES - N_EDGES
    spread = (jnp.arange(pad, dtype=jnp.int32) * 37) % N_NODES
    row = jnp.concatenate(
        [edge_index[0].astype(jnp.int32), spread]).reshape(-1, CHUNK)
    col = jnp.concatenate(
        [edge_index[1].astype(jnp.int32), spread]).reshape(-1, CHUNK)
    w = jnp.concatenate(
        [edge_weight.astype(jnp.float32),
         jnp.zeros((pad,), jnp.float32)]).reshape(-1, CHUNK)

    h1 = _mm(x, W1, b1, relu_in=False)
    p1 = _spmm(h1, row, col, w)
    h2 = _mm(p1, W2, b2, relu_in=True)
    return _spmm(h2, row, col, w)


# speedup vs baseline: 1.4005x; 1.0080x over previous
import jax
import jax.numpy as jnp
from jax import lax
from jax.experimental import pallas as pl
from jax.experimental.pallas import tpu as pltpu
from jax.experimental.pallas import tpu_sc as plsc

N_NODES = 10000
DIM = 128
HDIM = DIM // 2
N_EDGES = 320000

NC, NS = 2, 16
CHUNK = 64
CPT = 320
PAD_EDGES = NS * CPT * CHUNK
PHASE = 160
ROWS_PER_SUB = 624
HVREGS = HDIM // 16
NBUF = 8


def _spmm_body(h_hbm, row_hbm, col_hbm, w_hbm, out_hbm,
               acc, colv, ridxv, wv, rows, scaled, gsems, ssems):
    c = lax.axis_index("c")
    s = lax.axis_index("s")
    base_row = pl.multiple_of(s * ROWS_PER_SUB, 8)
    hsrc = h_hbm.at[c]

    for phase in range(CPT // PHASE):
        pbase = s * CPT + phase * PHASE
        pltpu.sync_copy(col_hbm.at[pl.ds(pbase, PHASE)], colv)
        pltpu.sync_copy(row_hbm.at[pl.ds(pbase, PHASE)], ridxv)
        pltpu.sync_copy(w_hbm.at[pl.ds(pbase, PHASE)], wv)

        for b in range(NBUF):
            pltpu.async_copy(hsrc.at[colv.at[b]], rows.at[b], gsems.at[b])

        if phase == 0:
            def _zero_row(e, _):
                for j in range(HVREGS):
                    scaled[0, e, 16 * j:16 * (j + 1)] = jnp.zeros(
                        (16,), jnp.float32)
                return 0
            lax.fori_loop(0, CHUNK, _zero_row, 0)
            for k in range(10):
                pltpu.sync_copy(scaled.at[0, pl.ds(0, CHUNK), :],
                                acc.at[pl.ds(base_row + CHUNK * k, CHUNK), :])
            plsc.subcore_barrier()

        def _step(t, b):
            i = NBUF * t + b
            sb = b % 2
            pltpu.make_async_copy(hsrc.at[colv.at[i]], rows.at[b],
                                  gsems.at[b]).wait()

            @pl.when(i >= 2)
            def _():
                pltpu.make_async_copy(scaled.at[sb], acc.at[ridxv.at[i - 2]],
                                      ssems.at[sb]).wait()

            def _scale_group(g, _):
                wg = wv[i, pl.ds(g * 16, 16)]
                for e in range(16):
                    wspl = jnp.broadcast_to(wg[e], (16,))
                    idx = g * 16 + e
                    for j in range(HVREGS):
                        sl = slice(16 * j, 16 * (j + 1))
                        scaled[sb, idx, sl] = rows[b, idx, sl] * wspl
                return 0
            lax.fori_loop(0, CHUNK // 16, _scale_group, 0)
            pltpu.async_copy(scaled.at[sb], acc.at[ridxv.at[i]],
                             ssems.at[sb], add=True)

            @pl.when(i + NBUF < PHASE)
            def _():
                pltpu.async_copy(hsrc.at[colv.at[i + NBUF]], rows.at[b],
                                 gsems.at[b])

        def _loop_body(t, _):
            for b in range(NBUF):
                _step(t, b)
            return 0
        lax.fori_loop(0, PHASE // NBUF, _loop_body, 0)

        for i in (PHASE - 2, PHASE - 1):
            pltpu.make_async_copy(scaled.at[i % 2], acc.at[ridxv.at[i]],
                                  ssems.at[i % 2]).wait()

    plsc.subcore_barrier()

    col_base = pl.multiple_of(c * HDIM, 8)

    @pl.when(s < NS - 1)
    def _():
        pltpu.sync_copy(acc.at[pl.ds(base_row, ROWS_PER_SUB), :],
                        out_hbm.at[pl.ds(base_row, ROWS_PER_SUB),
                                   pl.ds(col_base, HDIM)])

    @pl.when(s == NS - 1)
    def _():
        last = (NS - 1) * ROWS_PER_SUB
        pltpu.sync_copy(acc.at[pl.ds(last, N_NODES - last), :],
                        out_hbm.at[pl.ds(last, N_NODES - last),
                                   pl.ds(col_base, HDIM)])


@jax.jit
def _spmm(h2, row, col, w):
    mesh = plsc.VectorSubcoreMesh(core_axis_name="c", subcore_axis_name="s")
    run = pl.kernel(
        _spmm_body,
        out_type=jax.ShapeDtypeStruct((N_NODES, DIM), jnp.float32),
        mesh=mesh,
        compiler_params=pltpu.CompilerParams(use_tc_tiling_on_sc=False),
        scratch_types=[
            pltpu.VMEM_SHARED((N_NODES, HDIM), jnp.float32),
            pltpu.VMEM((PHASE, CHUNK), jnp.int32),
            pltpu.VMEM((PHASE, CHUNK), jnp.int32),
            pltpu.VMEM((PHASE, CHUNK), jnp.float32),
            pltpu.VMEM((NBUF, CHUNK, HDIM), jnp.float32),
            pltpu.VMEM((2, CHUNK, HDIM), jnp.float32),
            pltpu.SemaphoreType.DMA((NBUF,)),
            pltpu.SemaphoreType.DMA((2,)),
        ],
    )
    return run(h2, row, col, w)


ROW_BLK = 5000


def _mm_body(x_ref, w_ref, b_ref, o_ref, *, relu_in):
    xin = x_ref[...]
    if relu_in:
        xin = jnp.maximum(xin, 0.0)
    h = jnp.dot(xin, w_ref[...], preferred_element_type=jnp.float32) \
        + b_ref[...]
    o_ref[0] = h[:, :HDIM]
    o_ref[1] = h[:, HDIM:]


def _mm(x, W, b, relu_in):
    import functools
    return pl.pallas_call(
        functools.partial(_mm_body, relu_in=relu_in),
        grid=(N_NODES // ROW_BLK,),
        in_specs=[
            pl.BlockSpec((ROW_BLK, DIM), lambda i: (i, 0)),
            pl.BlockSpec((DIM, DIM), lambda i: (0, 0)),
            pl.BlockSpec((1, DIM), lambda i: (0, 0)),
        ],
        out_specs=pl.BlockSpec((NC, ROW_BLK, HDIM), lambda i: (0, i, 0)),
        out_shape=jax.ShapeDtypeStruct((NC, N_NODES, HDIM), jnp.float32),
    )(x, W, b.reshape(1, DIM))


def kernel(x, edge_index, edge_weight, W1, b1, W2, b2):
    pad = PAD_EDGES - N_EDGES
    spread = (jnp.arange(pad, dtype=jnp.int32) * 37) % N_NODES
    row = jnp.concatenate(
        [edge_index[0].astype(jnp.int32), spread]).reshape(-1, CHUNK)
    col = jnp.concatenate(
        [edge_index[1].astype(jnp.int32), spread]).reshape(-1, CHUNK)
    w = jnp.concatenate(
        [edge_weight.astype(jnp.float32),
         jnp.zeros((pad,), jnp.float32)]).reshape(-1, CHUNK)

    h1 = _mm(x, W1, b1, relu_in=False)
    p1 = _spmm(h1, row, col, w)
    h2 = _mm(p1, W2, b2, relu_in=True)
    return _spmm(h2, row, col, w)
